# baseline (speedup 1.0000x reference)
import functools

import jax
import jax.numpy as jnp
from jax import lax
from jax.experimental import pallas as pl
from jax.experimental.pallas import tpu as pltpu
from jax.experimental.pallas import tpu_sc as plsc

_CHUNK = 128


@functools.lru_cache(maxsize=None)
def _build_gather(n_tokens: int, vocab: int, dim: int):
    info = plsc.get_sparse_core_info()
    num_workers = info.num_cores * info.num_subcores
    assert n_tokens % (num_workers * _CHUNK) == 0
    per_w = n_tokens // num_workers
    n_chunks = per_w // _CHUNK

    mesh = plsc.VectorSubcoreMesh(core_axis_name="c", subcore_axis_name="s")

    @functools.partial(
        pl.kernel,
        mesh=mesh,
        out_type=jax.ShapeDtypeStruct((n_tokens, dim), jnp.float32),
        scratch_types=[
            pltpu.VMEM((n_chunks, _CHUNK), jnp.int32),
            pltpu.VMEM((per_w, dim), jnp.float32),
            pltpu.SemaphoreType.DMA,
        ],
        compiler_params=pltpu.CompilerParams(use_tc_tiling_on_sc=False),
    )
    def emb_kernel(ids_hbm, tab_hbm, out_hbm, idx_v, rows_v, sem):
        wid = lax.axis_index("s") * info.num_cores + lax.axis_index("c")
        base = wid * per_w
        pltpu.sync_copy(ids_hbm.at[pl.ds(wid * n_chunks, n_chunks)], idx_v)
        copies = []
        for j in range(n_chunks):
            copies.append(
                pltpu.async_copy(
                    tab_hbm.at[idx_v.at[j]],
                    rows_v.at[pl.ds(j * _CHUNK, _CHUNK)],
                    sem,
                )
            )
        for c in copies:
            c.wait()
        pltpu.sync_copy(rows_v, out_hbm.at[pl.ds(base, per_w)])

    return emb_kernel


def kernel(input_ids, embedding_table):
    batch, seq = input_ids.shape
    vocab, dim = embedding_table.shape
    n_tokens = batch * seq
    ids_flat = input_ids.reshape(n_tokens // _CHUNK, _CHUNK).astype(jnp.int32)
    out = _build_gather(n_tokens, vocab, dim)(ids_flat, embedding_table)
    return out.reshape(batch, seq, dim)


# speedup vs baseline: 1.4379x; 1.4379x over previous
import functools

import jax
import jax.numpy as jnp
from jax import lax
from jax.experimental import pallas as pl
from jax.experimental.pallas import tpu as pltpu
from jax.experimental.pallas import tpu_sc as plsc

_L = 16
_GROUP = 256
_BLK = 128
_CHUNK = 2048
_PCAP = 2048
_NSLOT = 4


def _cnt(mask):
    return jnp.sum(mask.astype(jnp.int32))


@functools.lru_cache(maxsize=None)
def _build(n_tokens: int, vocab: int, dim: int):
    info = plsc.get_sparse_core_info()
    nw = info.num_cores * info.num_subcores
    n_groups = (vocab + _GROUP - 1) // _GROUP
    n_chunks = n_tokens // _CHUNK
    last_cols = vocab - (n_groups - 1) * _GROUP

    mesh = plsc.VectorSubcoreMesh(core_axis_name="c", subcore_axis_name="s")

    @functools.partial(
        pl.kernel,
        mesh=mesh,
        out_type=jax.ShapeDtypeStruct((n_tokens + _L, 2 * dim), jnp.float32),
        scratch_types=[
            pltpu.VMEM((2, _CHUNK), jnp.int32),
            pltpu.VMEM((n_tokens + _L,), jnp.int32),
            pltpu.VMEM((n_tokens + _L,), jnp.int32),
            pltpu.VMEM((2, 2, dim, _BLK), jnp.float32),
            pltpu.VMEM((_PCAP + _L,), jnp.int32),
            pltpu.VMEM((_PCAP + _L,), jnp.int32),
            pltpu.VMEM((_NSLOT, _L, 2 * dim), jnp.float32),
            pltpu.VMEM((_NSLOT, _L), jnp.int32),
            pltpu.SemaphoreType.DMA,
            pltpu.SemaphoreType.DMA,
            pltpu.SemaphoreType.DMA,
        ],
        compiler_params=pltpu.CompilerParams(
            use_tc_tiling_on_sc=True, needs_layout_passes=False),
    )
    def emb_kernel(ids_hbm, tab_hbm, tail_hbm, out_hbm, idsbuf, rowl, tokl,
                   blocks, pend_r, pend_t, stage, scat_idx, sem_i, sem_b,
                   sem_s):
        wid = lax.axis_index("s") * info.num_cores + lax.axis_index("c")
        gs = wid * n_groups // nw
        ge = (wid + 1) * n_groups // nw
        lane = lax.iota(jnp.int32, _L)

        def blk_copies(g, par):
            c0 = g * _GROUP
            full = [
                pltpu.make_async_copy(
                    tab_hbm.at[:, pl.ds(c0, _BLK)], blocks.at[par, 0], sem_b),
                pltpu.make_async_copy(
                    tab_hbm.at[:, pl.ds(c0 + _BLK, _BLK)], blocks.at[par, 1],
                    sem_b),
            ]
            part = [
                pltpu.make_async_copy(tail_hbm, blocks.at[par, 0], sem_b),
            ]
            return full, part

        def issue_group(g, par):
            full, part = blk_copies(g, par)

            @pl.when(g < n_groups - 1)
            def _():
                for c in full:
                    c.start()

            @pl.when(g == n_groups - 1)
            def _():
                part[0].start()

        def wait_group(g, par):
            full, part = blk_copies(g, par)

            @pl.when(g < n_groups - 1)
            def _():
                for c in full:
                    c.wait()

            @pl.when(g == n_groups - 1)
            def _():
                part[0].wait()

        issue_group(gs, 0)
        pltpu.make_async_copy(
            ids_hbm.at[pl.ds(0, _CHUNK)], idsbuf.at[0], sem_i).start()
        for s in range(_NSLOT):
            scat_idx[s, :] = jnp.full((_L,), n_tokens, jnp.int32) + lane
            pltpu.make_async_copy(
                stage.at[s], out_hbm.at[scat_idx.at[s]], sem_s).start()

        lo = gs * _GROUP
        hi = ge * _GROUP

        def chunk_body(j, cnt):
            @pl.when(j + 1 < n_chunks)
            def _():
                pltpu.make_async_copy(
                    ids_hbm.at[pl.ds((j + 1) * _CHUNK, _CHUNK)],
                    idsbuf.at[(j + 1) % 2], sem_i).start()

            pltpu.make_async_copy(
                ids_hbm.at[pl.ds(j * _CHUNK, _CHUNK)],
                idsbuf.at[j % 2], sem_i).wait()

            def vec_body(k, c):
                v = idsbuf[j % 2, pl.ds(k * _L, _L)]
                pos = j * _CHUNK + k * _L + lane
                m = jnp.logical_and(v >= lo, v < hi)
                plsc.store_compressed(rowl.at[pl.ds(c, _L)], v, mask=m)
                plsc.store_compressed(tokl.at[pl.ds(c, _L)], pos, mask=m)
                return c + _cnt(m)

            return lax.fori_loop(0, _CHUNK // _L, vec_body, cnt)

        cnt = lax.fori_loop(0, n_chunks, chunk_body, jnp.int32(0))
        rowl[pl.ds(cnt, _L)] = jnp.full((_L,), 1 << 30, jnp.int32)
        n_vecs = (cnt + _L - 1) // _L

        def drain(pcnt, slot0, par):
            par_s = jnp.full((_L,), par, jnp.int32)

            def d_body(k, slotc):
                slot = slotc % _NSLOT
                slot_s = jnp.full((_L,), slot, jnp.int32)
                pltpu.make_async_copy(
                    stage.at[slot], out_hbm.at[scat_idx.at[slot]],
                    sem_s).wait()
                pt = pend_t[pl.ds(k * _L, _L)]
                pr = pend_r[pl.ds(k * _L, _L)]
                valid = (k * _L + lane) < pcnt
                ptv = jnp.where(valid, pt, jnp.full((_L,), n_tokens,
                                                    jnp.int32) + lane)
                prv = jnp.where(valid, pr, 0)
                jv = lax.shift_right_logical(prv, 7)
                rlv = lax.bitwise_and(prv, _BLK - 1)
                scat_idx[slot, :] = ptv

                def c_body(ci, _):
                    for u in range(4):
                        c = ci * 4 + u
                        c_s = jnp.full((_L,), c, jnp.int32)
                        vals = plsc.load_gather(
                            blocks, [par_s, jv, c_s, rlv])
                        plsc.store_scatter(
                            stage, [slot_s, lane, c_s], vals)
                    return 0

                lax.fori_loop(0, dim // 4, c_body, 0)
                pltpu.make_async_copy(
                    stage.at[slot], out_hbm.at[scat_idx.at[slot]],
                    sem_s).start()
                return slotc + 1

            n16 = (pcnt + _L - 1) // _L
            return lax.fori_loop(0, n16, d_body, slot0)

        def group_body(g, carry):
            slotc = carry
            par = (g - gs) % 2

            @pl.when(g + 1 < ge)
            def _():
                issue_group(g + 1, (g + 1 - gs) % 2)

            wait_group(g, par)
            glo = g * _GROUP

            def scan_body(k, sc):
                pc, slotc2 = sc
                rv = rowl[pl.ds(k * _L, _L)]
                tv = tokl[pl.ds(k * _L, _L)]
                m = jnp.logical_and(rv >= glo, rv < glo + _GROUP)
                plsc.store_compressed(
                    pend_r.at[pl.ds(pc, _L)], rv - glo, mask=m)
                plsc.store_compressed(
                    pend_t.at[pl.ds(pc, _L)], tv, mask=m)
                pc2 = pc + _cnt(m)

                return lax.cond(
                    pc2 >= _PCAP - _L,
                    lambda: (jnp.int32(0), drain(pc2, slotc2, par)),
                    lambda: (pc2, slotc2),
                )

            pc, slotc = lax.fori_loop(
                0, n_vecs, scan_body, (jnp.int32(0), slotc))
            slotc = drain(pc, slotc, par)
            return slotc

        slotc = lax.fori_loop(gs, ge, group_body, jnp.int32(0))

        def final_body(s, slotc2):
            slot = slotc2 % _NSLOT
            pltpu.make_async_copy(
                stage.at[slot], out_hbm.at[scat_idx.at[slot]], sem_s).wait()
            return slotc2 + 1

        lax.fori_loop(0, _NSLOT, final_body, slotc)

    return emb_kernel


def kernel(input_ids, embedding_table):
    batch, seq = input_ids.shape
    vocab, dim = embedding_table.shape
    n_tokens = batch * seq
    tab_t = jnp.transpose(embedding_table)
    last_cols = vocab - (vocab // _BLK) * _BLK
    tail = jnp.pad(tab_t[:, vocab - last_cols:],
                   ((0, 0), (0, _BLK - last_cols)))
    ids_flat = input_ids.reshape(n_tokens).astype(jnp.int32)
    out = _build(n_tokens, vocab, dim)(ids_flat, tab_t, tail)
    return out[:n_tokens, :dim].reshape(batch, seq, dim)


# speedup vs baseline: 1.4467x; 1.0061x over previous
import functools

import jax
import jax.numpy as jnp
from jax import lax
from jax.experimental import pallas as pl
from jax.experimental.pallas import tpu as pltpu
from jax.experimental.pallas import tpu_sc as plsc

_L = 16
_GROUP = 256
_BLK = 128
_CHUNK = 2048
_PCAP = 2048
_NSLOT = 4
_NSUB = 16
_GSUB = 8


def _popcnt(mask):
    return plsc.all_reduce_population_count(mask)[0]


@functools.lru_cache(maxsize=None)
def _build(n_tokens: int, vocab: int, dim: int):
    info = plsc.get_sparse_core_info()
    nw = info.num_cores * info.num_subcores
    n_groups = (vocab + _GROUP - 1) // _GROUP
    n_chunks = n_tokens // _CHUNK

    mesh = plsc.VectorSubcoreMesh(core_axis_name="c", subcore_axis_name="s")

    @functools.partial(
        pl.kernel,
        mesh=mesh,
        out_type=jax.ShapeDtypeStruct((n_tokens + _L, 2 * dim), jnp.float32),
        scratch_types=[
            pltpu.VMEM((2, _CHUNK), jnp.int32),
            pltpu.VMEM((n_tokens + _L,), jnp.int32),
            pltpu.VMEM((n_tokens + _L,), jnp.int32),
            pltpu.VMEM((2, 2, dim, _BLK), jnp.float32),
            pltpu.VMEM((_PCAP + _L,), jnp.int32),
            pltpu.VMEM((_NSLOT, _L, 2 * dim), jnp.float32),
            pltpu.VMEM((_NSLOT, _L), jnp.int32),
            pltpu.SemaphoreType.DMA,
            pltpu.SemaphoreType.DMA,
            pltpu.SemaphoreType.DMA,
        ],
        compiler_params=pltpu.CompilerParams(
            use_tc_tiling_on_sc=True, needs_layout_passes=False),
    )
    def emb_kernel(ids_hbm, tab_hbm, tail_hbm, out_hbm, idsbuf, slabl, subl,
                   blocks, pend, stage, scat_idx, sem_i, sem_b, sem_s):
        wid = lax.axis_index("s") * info.num_cores + lax.axis_index("c")
        gs = wid * n_groups // nw
        ge = (wid + 1) * n_groups // nw
        lane = lax.iota(jnp.int32, _L)

        def blk_copies(g, par):
            c0 = g * _GROUP
            full = [
                pltpu.make_async_copy(
                    tab_hbm.at[:, pl.ds(c0, _BLK)], blocks.at[par, 0], sem_b),
                pltpu.make_async_copy(
                    tab_hbm.at[:, pl.ds(c0 + _BLK, _BLK)], blocks.at[par, 1],
                    sem_b),
            ]
            part = [
                pltpu.make_async_copy(tail_hbm, blocks.at[par, 0], sem_b),
            ]
            return full, part

        def issue_group(g, par):
            full, part = blk_copies(g, par)

            @pl.when(g < n_groups - 1)
            def _():
                for c in full:
                    c.start()

            @pl.when(g == n_groups - 1)
            def _():
                part[0].start()

        def wait_group(g, par):
            full, part = blk_copies(g, par)

            @pl.when(g < n_groups - 1)
            def _():
                for c in full:
                    c.wait()

            @pl.when(g == n_groups - 1)
            def _():
                part[0].wait()

        issue_group(gs, 0)
        pltpu.make_async_copy(
            ids_hbm.at[pl.ds(0, _CHUNK)], idsbuf.at[0], sem_i).start()
        for s in range(_NSLOT):
            scat_idx[s, :] = jnp.full((_L,), n_tokens, jnp.int32) + lane
            pltpu.make_async_copy(
                stage.at[s], out_hbm.at[scat_idx.at[s]], sem_s).start()

        lo = gs * _GROUP
        hi = ge * _GROUP

        def chunk_body(j, cnt):
            @pl.when(j + 1 < n_chunks)
            def _():
                pltpu.make_async_copy(
                    ids_hbm.at[pl.ds((j + 1) * _CHUNK, _CHUNK)],
                    idsbuf.at[(j + 1) % 2], sem_i).start()

            pltpu.make_async_copy(
                ids_hbm.at[pl.ds(j * _CHUNK, _CHUNK)],
                idsbuf.at[j % 2], sem_i).wait()

            def vec_body(k, c):
                v = idsbuf[j % 2, pl.ds(k * _L, _L)]
                pos = j * _CHUNK + k * _L + lane
                m = jnp.logical_and(v >= lo, v < hi)
                packed = (v - lo) * 65536 + pos
                plsc.store_compressed(slabl.at[pl.ds(c, _L)], packed, mask=m)
                return c + _popcnt(m)

            return lax.fori_loop(0, _CHUNK // _L, vec_body, cnt)

        cnt = lax.fori_loop(0, n_chunks, chunk_body, jnp.int32(0))

        n_vecs = (cnt + _L - 1) // _L
        offs = [jnp.int32(0)]
        for b in range(_NSUB):
            b_lo = b * _GSUB * _GROUP << 16
            b_hi = (b + 1) * _GSUB * _GROUP << 16

            def part_body(k, c, b_lo=b_lo, b_hi=b_hi):
                v = slabl[pl.ds(k * _L, _L)]
                valid = (k * _L + lane) < cnt
                m = jnp.logical_and(v >= b_lo, valid)
                if b < _NSUB - 1:
                    m = jnp.logical_and(m, v < b_hi)
                plsc.store_compressed(subl.at[pl.ds(c, _L)], v, mask=m)
                return c + _popcnt(m)

            offs.append(lax.fori_loop(0, n_vecs, part_body, offs[-1]))

        def drain(pcnt, slot0, par, grel_base):
            par_s = jnp.full((_L,), par, jnp.int32)

            def d_body(k, slotc):
                slot = slotc % _NSLOT
                slot_s = jnp.full((_L,), slot, jnp.int32)
                pltpu.make_async_copy(
                    stage.at[slot], out_hbm.at[scat_idx.at[slot]],
                    sem_s).wait()
                pv = pend[pl.ds(k * _L, _L)]
                valid = (k * _L + lane) < pcnt
                ptv = jnp.where(
                    valid, jnp.bitwise_and(pv, 0xFFFF),
                    jnp.full((_L,), n_tokens, jnp.int32) + lane)
                prv = jnp.where(
                    valid, jnp.right_shift(pv, 16) - grel_base, 0)
                jv = jnp.right_shift(prv, 7)
                rlv = jnp.bitwise_and(prv, _BLK - 1)
                scat_idx[slot, :] = ptv

                def c_body(ci, _):
                    for u in range(4):
                        c_s = ci * 4 + u
                        c_v = jnp.full((_L,), c_s, jnp.int32)
                        vals = plsc.load_gather(
                            blocks, [par_s, jv, c_v, rlv])
                        plsc.store_scatter(
                            stage, [slot_s, lane, c_v], vals)
                    return 0

                lax.fori_loop(0, dim // 4, c_body, 0)
                pltpu.make_async_copy(
                    stage.at[slot], out_hbm.at[scat_idx.at[slot]],
                    sem_s).start()
                return slotc + 1

            n16 = (pcnt + _L - 1) // _L
            return lax.fori_loop(0, n16, d_body, slot0)

        slotc = jnp.int32(0)
        for b in range(_NSUB):
            s_lo, s_hi = offs[b], offs[b + 1]
            gb = gs + b * _GSUB
            nb = jnp.clip(ge - gb, 0, _GSUB)

            def group_body(i, slotc2, b=b, s_lo=s_lo, s_hi=s_hi, gb=gb):
                g = gb + i
                par = (g - gs) % 2

                @pl.when(g + 1 < ge)
                def _():
                    issue_group(g + 1, (g + 1 - gs) % 2)

                wait_group(g, par)
                grel_base = (g - gs) * _GROUP
                p_lo = grel_base * 65536
                p_hi = (grel_base + _GROUP) * 65536

                def scan_body(k, sc):
                    pc, slotc3 = sc
                    v = subl[pl.ds(s_lo + k * _L, _L)]
                    valid = (s_lo + k * _L + lane) < s_hi
                    m = jnp.logical_and(
                        jnp.logical_and(v >= p_lo, v < p_hi), valid)
                    plsc.store_compressed(pend.at[pl.ds(pc, _L)], v, mask=m)
                    pc2 = pc + _popcnt(m)

                    return lax.cond(
                        pc2 >= _PCAP - _L,
                        lambda: (jnp.int32(0),
                                 drain(pc2, slotc3, par, grel_base)),
                        lambda: (pc2, slotc3),
                    )

                nv = (s_hi - s_lo + _L - 1) // _L
                pc, slotc2 = lax.fori_loop(
                    0, nv, scan_body, (jnp.int32(0), slotc2))
                return drain(pc, slotc2, par, grel_base)

            slotc = lax.fori_loop(0, nb, group_body, slotc)

        def final_body(s, slotc2):
            slot = slotc2 % _NSLOT
            pltpu.make_async_copy(
                stage.at[slot], out_hbm.at[scat_idx.at[slot]], sem_s).wait()
            return slotc2 + 1

        lax.fori_loop(0, _NSLOT, final_body, slotc)

    return emb_kernel


def kernel(input_ids, embedding_table):
    batch, seq = input_ids.shape
    vocab, dim = embedding_table.shape
    n_tokens = batch * seq
    tab_t = jnp.transpose(embedding_table)
    last_cols = vocab - (vocab // _BLK) * _BLK
    tail = jnp.pad(tab_t[:, vocab - last_cols:],
                   ((0, 0), (0, _BLK - last_cols)))
    ids_flat = input_ids.reshape(n_tokens).astype(jnp.int32)
    out = _build(n_tokens, vocab, dim)(ids_flat, tab_t, tail)
    return out[:n_tokens, :dim].reshape(batch, seq, dim)
